# baseline (speedup 1.0000x reference)
import functools

import jax
import jax.numpy as jnp
from jax import lax
from jax.experimental import pallas as pl
from jax.experimental.pallas import tpu as pltpu
from jax.experimental.pallas import tpu_sc as plsc

D = 64
B = 16384
NC, NS, L = 2, 16, 16
NW = NC * NS
TOT = 2 * B
PER_W = TOT // NW
CH = 128
NCH = PER_W // CH
NG = PER_W // L

_mesh = plsc.VectorSubcoreMesh(core_axis_name="c", subcore_axis_name="s")


@functools.partial(
    pl.kernel,
    mesh=_mesh,
    out_type=jax.ShapeDtypeStruct((NW, PER_W), jnp.float32),
    scratch_types=[
        pltpu.VMEM((NCH, CH), jnp.int32),
        pltpu.VMEM((PER_W, D), jnp.float32),
        pltpu.VMEM((PER_W,), jnp.float32),
        pltpu.VMEM((L * L,), jnp.float32),
        pltpu.SemaphoreType.DMA,
    ],
    compiler_params=pltpu.CompilerParams(
        needs_layout_passes=False, use_tc_tiling_on_sc=False),
)
def _row_sums_sc(idx_hbm, emb_hbm, out_hbm, idx_v, rows_v, sums_v, t_buf, sem):
    wid = lax.axis_index("s") * NC + lax.axis_index("c")
    pltpu.sync_copy(idx_hbm.at[wid], idx_v)

    copies = [
        pltpu.async_copy(emb_hbm.at[idx_v.at[j]],
                         rows_v.at[pl.ds(j * CH, CH)], sem)
        for j in range(NCH)
    ]
    for c in copies:
        c.wait()

    def body(g, carry):
        base = g * L
        for i in range(L):
            r = base + i
            v01 = rows_v[r, pl.ds(0, 16)] + rows_v[r, pl.ds(16, 16)]
            v23 = rows_v[r, pl.ds(32, 16)] + rows_v[r, pl.ds(48, 16)]
            t_buf[pl.ds(i * L, L)] = v01 + v23
        lanes = lax.iota(jnp.int32, L) * L
        acc = jnp.zeros((L,), jnp.float32)
        for dd in range(L):
            acc = acc + plsc.load_gather(t_buf, [lanes + dd])
        sums_v[pl.ds(base, L)] = acc
        return carry

    lax.fori_loop(0, NG, body, 0)
    pltpu.sync_copy(sums_v, out_hbm.at[wid])


def _loss_body(s_ref, o_ref):
    s = s_ref[...]
    sp = s[: B // 128, :]
    sn = s[B // 128:, :]
    pos_loss = jnp.log1p(jnp.exp(-sp))
    neg_loss = jnp.log1p(jnp.exp(sn))
    o_ref[...] = ((jnp.sum(pos_loss) + jnp.sum(neg_loss)) / B).reshape(1, 1)


_loss_tc = pl.pallas_call(
    _loss_body,
    out_shape=jax.ShapeDtypeStruct((1, 1), jnp.float32),
)


def kernel(pos_words, neg_words, embeddings):
    idx = jnp.concatenate([pos_words, neg_words]).astype(jnp.int32)
    idx = idx.reshape(NW, NCH, CH)
    sums = _row_sums_sc(idx, embeddings)
    loss = _loss_tc(sums.reshape(TOT // 128, 128))
    return loss[0, 0]
